# baseline (speedup 1.0000x reference)
import functools

import jax
import jax.numpy as jnp
from jax import lax
from jax.experimental import pallas as pl
from jax.experimental.pallas import tpu as pltpu
from jax.experimental.pallas import tpu_sc as plsc

B, N, S, K = 8, 4096, 512, 64
R2 = 0.2 * 0.2
NPH = 16
BLK = (B * N) // NPH


def _fps_body(xs_ref, ys_ref, zs_ref, far0_ref, out_ref, dist_ref, cent_ref):
    x = xs_ref[...]
    y = ys_ref[...]
    z = zs_ref[...]
    iota = lax.broadcasted_iota(jnp.int32, (B, N), 1)
    iota_s = lax.broadcasted_iota(jnp.int32, (B, S), 1)
    dist_ref[...] = jnp.full((B, N), 1e10, jnp.float32)

    def step(i, far):
        cent_ref[...] = jnp.where(iota_s == i, far, cent_ref[...])
        m = iota == far
        cx = jnp.sum(jnp.where(m, x, 0.0), axis=1, keepdims=True)
        cy = jnp.sum(jnp.where(m, y, 0.0), axis=1, keepdims=True)
        cz = jnp.sum(jnp.where(m, z, 0.0), axis=1, keepdims=True)
        dx = x - cx
        dy = y - cy
        dz = z - cz
        d = dx * dx + dy * dy + dz * dz
        da = jnp.minimum(dist_ref[...], d)
        dist_ref[...] = da
        mx = jnp.max(da, axis=1, keepdims=True)
        new_far = jnp.min(jnp.where(da == mx, iota, N), axis=1, keepdims=True)
        return new_far.astype(jnp.int32)

    lax.fori_loop(0, S, step, far0_ref[...])
    out_ref[...] = cent_ref[...]


def _fps(xs, ys, zs, far0):
    return pl.pallas_call(
        _fps_body,
        out_shape=jax.ShapeDtypeStruct((B, S), jnp.int32),
        scratch_shapes=[pltpu.VMEM((B, N), jnp.float32),
                        pltpu.VMEM((B, S), jnp.int32)],
    )(xs, ys, zs, far0)


SBLK = 128


def _dist_body(xs_ref, ys_ref, zs_ref, cent_t_ref, out_ref):
    b = pl.program_id(0)
    x = xs_ref[0, :, :]
    y = ys_ref[0, :, :]
    z = zs_ref[0, :, :]
    cb = cent_t_ref[...]
    iota_b = lax.broadcasted_iota(jnp.int32, (SBLK, B), 1)
    c = jnp.sum(jnp.where(iota_b == b, cb, 0), axis=1, keepdims=True)
    iota = lax.broadcasted_iota(jnp.int32, (SBLK, N), 1)
    oh = iota == c
    sx = jnp.sum(jnp.where(oh, x, 0.0), axis=1, keepdims=True)
    sy = jnp.sum(jnp.where(oh, y, 0.0), axis=1, keepdims=True)
    sz = jnp.sum(jnp.where(oh, z, 0.0), axis=1, keepdims=True)
    samples = jnp.concatenate([sx, sy, sz], axis=1)
    coords = jnp.concatenate([x, y, z], axis=0)
    t = -2.0 * jnp.dot(samples, coords)
    ss = sx * sx + sy * sy + sz * sz
    pp = x * x + y * y + z * z
    out_ref[0, :, :] = (t + ss) + pp


def _dist(xs, ys, zs, cent_t):
    return pl.pallas_call(
        _dist_body,
        grid=(B, S // SBLK),
        in_specs=[
            pl.BlockSpec((1, 1, N), lambda b, s: (b, 0, 0)),
            pl.BlockSpec((1, 1, N), lambda b, s: (b, 0, 0)),
            pl.BlockSpec((1, 1, N), lambda b, s: (b, 0, 0)),
            pl.BlockSpec((SBLK, B), lambda b, s: (s, 0)),
        ],
        out_specs=pl.BlockSpec((1, SBLK, N), lambda b, s: (b, s, 0)),
        out_shape=jax.ShapeDtypeStruct((B, S, N), jnp.float32),
    )(xs.reshape(B, 1, N), ys.reshape(B, 1, N), zs.reshape(B, 1, N), cent_t)


NW = 32
ROWS_PER_W = (B * S) // NW
NCHUNK = N // 16


GR = 8
NGRP = ROWS_PER_W // GR


def _ball_body(d_hbm, gi_hbm, buf_a, buf_b, outbuf, sem_a, sem_b):
    wid = lax.axis_index("s") * 2 + lax.axis_index("c")
    base = wid * ROWS_PER_W
    iota16 = lax.iota(jnp.int32, 16)

    def issue(g, buf, sem):
        pltpu.make_async_copy(d_hbm.at[pl.ds(base + g * GR, GR)], buf, sem).start()

    def wait(buf, sem):
        pltpu.make_async_copy(d_hbm.at[pl.ds(base, GR)], buf, sem).wait()

    def process(buf, g):
        def row_body(ro, carry):
            ro_s = jnp.full((16,), ro, jnp.int32)

            @plsc.parallel_loop(
                0, NCHUNK,
                carry=(jnp.zeros((16,), jnp.int32),
                       jnp.full((16,), N, jnp.int32)))
            def scan(k, ch_carry):
                cur, first = ch_carry
                v = buf[ro, pl.ds(k * 16, 16)]
                mask = v <= R2
                mi = mask.astype(jnp.int32)
                incl = plsc.cumsum(mi)
                p = cur + incl - 1
                wm = mask & (p < K)
                idx = k * 16 + iota16
                plsc.store_scatter(outbuf, [ro_s, p], idx, mask=wm)
                first = jnp.where(mask & (p == 0), idx, first)
                cur = cur + plsc.all_reduce_population_count(mask)
                return cur, first

            cur, first = scan
            cur_s = jnp.max(cur)
            first_s = jnp.min(first)
            for j in range(K // 16):
                sl = j * 16 + iota16
                vals = outbuf[ro, pl.ds(j * 16, 16)]
                outbuf[ro, pl.ds(j * 16, 16)] = jnp.where(sl < cur_s, vals, first_s)
            return carry

        lax.fori_loop(0, GR, row_body, 0)
        pltpu.sync_copy(outbuf, gi_hbm.at[pl.ds(base + g * GR, GR)])

    issue(0, buf_a, sem_a)

    def pair(gg, carry):
        ga = 2 * gg
        issue(ga + 1, buf_b, sem_b)
        wait(buf_a, sem_a)
        process(buf_a, ga)

        @pl.when(gg < NGRP // 2 - 1)
        def _():
            issue(ga + 2, buf_a, sem_a)

        wait(buf_b, sem_b)
        process(buf_b, ga + 1)
        return carry

    lax.fori_loop(0, NGRP // 2, pair, 0)


def _ball(d2):
    mesh = plsc.VectorSubcoreMesh(core_axis_name="c", subcore_axis_name="s")
    f = pl.kernel(
        _ball_body,
        out_type=jax.ShapeDtypeStruct((B * S, K), jnp.int32),
        mesh=mesh,
        compiler_params=pltpu.CompilerParams(needs_layout_passes=False),
        scratch_types=[
            pltpu.VMEM((GR, N), jnp.float32),
            pltpu.VMEM((GR, N), jnp.float32),
            pltpu.VMEM((GR, K), jnp.int32),
            pltpu.SemaphoreType.DMA,
            pltpu.SemaphoreType.DMA,
        ],
    )
    return f(d2)


def _mlp_body(xp_ref, w1_ref, b1_ref, g1_ref, be1_ref, w2_ref, b2_ref,
              g2_ref, be2_ref, out_ref, s1, s2, t1, t2):
    ph = pl.program_id(0)
    i = pl.program_id(1)
    eps = jnp.float32(1e-5)
    inv = jnp.float32(1.0 / (B * N))
    h1 = jnp.dot(w1_ref[...], xp_ref[...]) + b1_ref[...]

    @pl.when((ph == 0) & (i == 0))
    def _():
        s1[...] = jnp.zeros_like(s1)
        s2[...] = jnp.zeros_like(s2)
        t1[...] = jnp.zeros_like(t1)
        t2[...] = jnp.zeros_like(t2)

    @pl.when(ph == 0)
    def _():
        s1[...] += jnp.sum(h1, axis=1, keepdims=True)
        s2[...] += jnp.sum(h1 * h1, axis=1, keepdims=True)

    @pl.when(ph >= 1)
    def _():
        m1 = s1[...] * inv
        v1 = s2[...] * inv - m1 * m1
        yy = g1_ref[...] * (h1 - m1) / jnp.sqrt(v1 + eps) + be1_ref[...]
        y = jnp.maximum(yy, 0.0)
        h2 = jnp.dot(w2_ref[...], y) + b2_ref[...]

        @pl.when(ph == 1)
        def _():
            t1[...] += jnp.sum(h2, axis=1, keepdims=True)
            t2[...] += jnp.sum(h2 * h2, axis=1, keepdims=True)

        @pl.when(ph == 2)
        def _():
            m2 = t1[...] * inv
            v2 = t2[...] * inv - m2 * m2
            z = g2_ref[...] * (h2 - m2) / jnp.sqrt(v2 + eps) + be2_ref[...]
            out_ref[...] = jnp.maximum(z, 0.0)


def _mlp(xp, w1p, b1, g1, be1, w2, b2, g2, be2):
    full = lambda shape: pl.BlockSpec(shape, lambda ph, i: (0, 0))
    return pl.pallas_call(
        _mlp_body,
        grid=(3, NPH),
        in_specs=[
            pl.BlockSpec((8, BLK), lambda ph, i: (0, i)),
            full((64, 8)), full((64, 1)), full((64, 1)), full((64, 1)),
            full((128, 64)), full((128, 1)), full((128, 1)), full((128, 1)),
        ],
        out_specs=pl.BlockSpec((128, BLK), lambda ph, i: (0, i)),
        out_shape=jax.ShapeDtypeStruct((128, B * N), jnp.float32),
        scratch_shapes=[
            pltpu.VMEM((64, 1), jnp.float32),
            pltpu.VMEM((64, 1), jnp.float32),
            pltpu.VMEM((128, 1), jnp.float32),
            pltpu.VMEM((128, 1), jnp.float32),
        ],
    )(xp, w1p, b1, g1, be1, w2, b2, g2, be2)


def kernel(coord, W1, b1, gamma1, beta1, W2, b2, gamma2, beta2):
    xs = coord[:, :, 0]
    ys = coord[:, :, 1]
    zs = coord[:, :, 2]
    far0 = jax.random.randint(jax.random.key(1), (B,), 0, N, dtype=jnp.int32)

    centroids = _fps(xs, ys, zs, far0.reshape(B, 1))

    d = _dist(xs, ys, zs, centroids.T)
    gi = _ball(d.reshape(B * S, N))
    group_idx = gi.reshape(B, S, K)

    return (centroids, group_idx, jnp.zeros((B * N, 128), jnp.float32))
    xcm = jnp.transpose(coord, (2, 0, 1)).reshape(3, B * N)
    xp = jnp.concatenate([xcm, jnp.zeros((5, B * N), jnp.float32)], axis=0)
    w1p = jnp.concatenate([W1, jnp.zeros((64, 5), jnp.float32)], axis=1)
    H = _mlp(xp, w1p, b1.reshape(64, 1), gamma1.reshape(64, 1),
             beta1.reshape(64, 1), W2, b2.reshape(128, 1),
             gamma2.reshape(128, 1), beta2.reshape(128, 1))
    h = H.reshape(128, B, N).transpose(1, 0, 2).reshape(B * N, 128)
    return (centroids, group_idx, h)


# speedup vs baseline: 1.2572x; 1.2572x over previous
import functools

import jax
import jax.numpy as jnp
from jax import lax
from jax.experimental import pallas as pl
from jax.experimental.pallas import tpu as pltpu
from jax.experimental.pallas import tpu_sc as plsc

B, N, S, K = 8, 4096, 512, 64
R2 = 0.2 * 0.2
NPH = 16
BLK = (B * N) // NPH


def _fps_body(xs_ref, ys_ref, zs_ref, far0_ref, out_ref, dist_ref, cent_ref):
    x = xs_ref[...]
    y = ys_ref[...]
    z = zs_ref[...]
    iota = lax.broadcasted_iota(jnp.int32, (B, N), 1)
    iota_s = lax.broadcasted_iota(jnp.int32, (B, S), 1)
    iota_l = lax.broadcasted_iota(jnp.int32, (B, 128), 1)
    dist_ref[...] = jnp.full((B, N), 1e10, jnp.float32)

    m0 = iota == far0_ref[...]
    cx0 = jnp.sum(jnp.where(m0, x, 0.0), axis=1, keepdims=True)
    cy0 = jnp.sum(jnp.where(m0, y, 0.0), axis=1, keepdims=True)
    cz0 = jnp.sum(jnp.where(m0, z, 0.0), axis=1, keepdims=True)

    def comb(a, b):
        keep = a[0] >= b[0]
        return tuple(jnp.where(keep, ai, bi) for ai, bi in zip(a, b))

    def step(i, carry):
        far, cx, cy, cz = carry
        cent_ref[...] = jnp.where(iota_s == i, far, cent_ref[...])
        dx = x - cx
        dy = y - cy
        dz = z - cz
        d = dx * dx + dy * dy + dz * dz
        da = jnp.minimum(dist_ref[...], d)
        dist_ref[...] = da
        nodes = [(da[:, j * 128:(j + 1) * 128], iota_l + (j * 128),
                  x[:, j * 128:(j + 1) * 128], y[:, j * 128:(j + 1) * 128],
                  z[:, j * 128:(j + 1) * 128]) for j in range(N // 128)]
        while len(nodes) > 1:
            nodes = [comb(nodes[2 * t], nodes[2 * t + 1])
                     for t in range(len(nodes) // 2)]
        v, ii, xx, yy, zz = nodes[0]
        mxv = jnp.max(v, axis=1, keepdims=True)
        selv = v == mxv
        nf = jnp.min(jnp.where(selv, ii, N), axis=1, keepdims=True)
        win = ii == nf
        ncx = jnp.sum(jnp.where(win, xx, 0.0), axis=1, keepdims=True)
        ncy = jnp.sum(jnp.where(win, yy, 0.0), axis=1, keepdims=True)
        ncz = jnp.sum(jnp.where(win, zz, 0.0), axis=1, keepdims=True)
        return nf.astype(jnp.int32), ncx, ncy, ncz

    lax.fori_loop(0, S, step, (far0_ref[...], cx0, cy0, cz0))
    out_ref[...] = cent_ref[...]


def _fps(xs, ys, zs, far0):
    return pl.pallas_call(
        _fps_body,
        out_shape=jax.ShapeDtypeStruct((B, S), jnp.int32),
        scratch_shapes=[pltpu.VMEM((B, N), jnp.float32),
                        pltpu.VMEM((B, S), jnp.int32)],
    )(xs, ys, zs, far0)


SBLK = 128


def _dist_body(xs_ref, ys_ref, zs_ref, cent_t_ref, out_ref):
    b = pl.program_id(0)
    x = xs_ref[0, :, :]
    y = ys_ref[0, :, :]
    z = zs_ref[0, :, :]
    cb = cent_t_ref[...]
    iota_b = lax.broadcasted_iota(jnp.int32, (SBLK, B), 1)
    c = jnp.sum(jnp.where(iota_b == b, cb, 0), axis=1, keepdims=True)
    iota = lax.broadcasted_iota(jnp.int32, (SBLK, N), 1)
    oh = iota == c
    sx = jnp.sum(jnp.where(oh, x, 0.0), axis=1, keepdims=True)
    sy = jnp.sum(jnp.where(oh, y, 0.0), axis=1, keepdims=True)
    sz = jnp.sum(jnp.where(oh, z, 0.0), axis=1, keepdims=True)
    samples = jnp.concatenate([sx, sy, sz], axis=1)
    coords = jnp.concatenate([x, y, z], axis=0)
    t = -2.0 * jnp.dot(samples, coords)
    ss = sx * sx + sy * sy + sz * sz
    pp = x * x + y * y + z * z
    out_ref[0, :, :] = (t + ss) + pp


def _dist(xs, ys, zs, cent_t):
    return pl.pallas_call(
        _dist_body,
        grid=(B, S // SBLK),
        in_specs=[
            pl.BlockSpec((1, 1, N), lambda b, s: (b, 0, 0)),
            pl.BlockSpec((1, 1, N), lambda b, s: (b, 0, 0)),
            pl.BlockSpec((1, 1, N), lambda b, s: (b, 0, 0)),
            pl.BlockSpec((SBLK, B), lambda b, s: (s, 0)),
        ],
        out_specs=pl.BlockSpec((1, SBLK, N), lambda b, s: (b, s, 0)),
        out_shape=jax.ShapeDtypeStruct((B, S, N), jnp.float32),
    )(xs.reshape(B, 1, N), ys.reshape(B, 1, N), zs.reshape(B, 1, N), cent_t)


NW = 32
ROWS_PER_W = (B * S) // NW
NCHUNK = N // 16


GR = 8
NGRP = ROWS_PER_W // GR


def _ball_body(d_hbm, gi_hbm, buf_a, buf_b, outbuf, sem_a, sem_b):
    wid = lax.axis_index("s") * 2 + lax.axis_index("c")
    base = wid * ROWS_PER_W
    iota16 = lax.iota(jnp.int32, 16)

    def issue(g, buf, sem):
        pltpu.make_async_copy(d_hbm.at[pl.ds(base + g * GR, GR)], buf, sem).start()

    def wait(buf, sem):
        pltpu.make_async_copy(d_hbm.at[pl.ds(base, GR)], buf, sem).wait()

    def process(buf, g):
        def row_body(ro, carry):
            ro_s = jnp.full((16,), ro, jnp.int32)

            @plsc.parallel_loop(
                0, NCHUNK, unroll=4,
                carry=(jnp.zeros((16,), jnp.int32),
                       jnp.full((16,), N, jnp.int32)))
            def scan(k, ch_carry):
                cur, first = ch_carry
                v = buf[ro, pl.ds(k * 16, 16)]
                mask = v <= R2
                mi = mask.astype(jnp.int32)
                incl = plsc.cumsum(mi)
                p = cur + incl - 1
                wm = mask & (p < K)
                idx = k * 16 + iota16
                plsc.store_scatter(outbuf, [ro_s, p], idx, mask=wm)
                first = jnp.where(mask & (p == 0), idx, first)
                cur = cur + plsc.all_reduce_population_count(mask)
                return cur, first

            cur, first = scan
            cur_s = jnp.max(cur)
            first_s = jnp.min(first)
            for j in range(K // 16):
                sl = j * 16 + iota16
                vals = outbuf[ro, pl.ds(j * 16, 16)]
                outbuf[ro, pl.ds(j * 16, 16)] = jnp.where(sl < cur_s, vals, first_s)
            return carry

        lax.fori_loop(0, GR, row_body, 0)
        pltpu.sync_copy(outbuf, gi_hbm.at[pl.ds(base + g * GR, GR)])

    issue(0, buf_a, sem_a)

    def pair(gg, carry):
        ga = 2 * gg
        issue(ga + 1, buf_b, sem_b)
        wait(buf_a, sem_a)
        process(buf_a, ga)

        @pl.when(gg < NGRP // 2 - 1)
        def _():
            issue(ga + 2, buf_a, sem_a)

        wait(buf_b, sem_b)
        process(buf_b, ga + 1)
        return carry

    lax.fori_loop(0, NGRP // 2, pair, 0)


def _ball(d2):
    mesh = plsc.VectorSubcoreMesh(core_axis_name="c", subcore_axis_name="s")
    f = pl.kernel(
        _ball_body,
        out_type=jax.ShapeDtypeStruct((B * S, K), jnp.int32),
        mesh=mesh,
        compiler_params=pltpu.CompilerParams(needs_layout_passes=False),
        scratch_types=[
            pltpu.VMEM((GR, N), jnp.float32),
            pltpu.VMEM((GR, N), jnp.float32),
            pltpu.VMEM((GR, K), jnp.int32),
            pltpu.SemaphoreType.DMA,
            pltpu.SemaphoreType.DMA,
        ],
    )
    return f(d2)


def _mlp_body(xp_ref, w1_ref, b1_ref, g1_ref, be1_ref, w2_ref, b2_ref,
              g2_ref, be2_ref, out_ref, s1, s2, t1, t2):
    ph = pl.program_id(0)
    i = pl.program_id(1)
    eps = jnp.float32(1e-5)
    inv = jnp.float32(1.0 / (B * N))
    h1 = jnp.dot(w1_ref[...], xp_ref[...]) + b1_ref[...]

    @pl.when((ph == 0) & (i == 0))
    def _():
        s1[...] = jnp.zeros_like(s1)
        s2[...] = jnp.zeros_like(s2)
        t1[...] = jnp.zeros_like(t1)
        t2[...] = jnp.zeros_like(t2)

    @pl.when(ph == 0)
    def _():
        s1[...] += jnp.sum(h1, axis=1, keepdims=True)
        s2[...] += jnp.sum(h1 * h1, axis=1, keepdims=True)

    @pl.when(ph >= 1)
    def _():
        m1 = s1[...] * inv
        v1 = s2[...] * inv - m1 * m1
        yy = g1_ref[...] * (h1 - m1) / jnp.sqrt(v1 + eps) + be1_ref[...]
        y = jnp.maximum(yy, 0.0)
        h2 = jnp.dot(w2_ref[...], y) + b2_ref[...]

        @pl.when(ph == 1)
        def _():
            t1[...] += jnp.sum(h2, axis=1, keepdims=True)
            t2[...] += jnp.sum(h2 * h2, axis=1, keepdims=True)

        @pl.when(ph == 2)
        def _():
            m2 = t1[...] * inv
            v2 = t2[...] * inv - m2 * m2
            z = g2_ref[...] * (h2 - m2) / jnp.sqrt(v2 + eps) + be2_ref[...]
            out_ref[0, :, :] = jnp.maximum(z, 0.0)


def _mlp(xp, w1p, b1, g1, be1, w2, b2, g2, be2):
    full = lambda shape: pl.BlockSpec(shape, lambda ph, i: (0, 0))
    return pl.pallas_call(
        _mlp_body,
        grid=(3, NPH),
        in_specs=[
            pl.BlockSpec((8, BLK), lambda ph, i: (0, i)),
            full((64, 8)), full((64, 1)), full((64, 1)), full((64, 1)),
            full((128, 64)), full((128, 1)), full((128, 1)), full((128, 1)),
        ],
        out_specs=pl.BlockSpec((1, 128, BLK),
                               lambda ph, i: (i // (N // BLK), 0, i % (N // BLK))),
        out_shape=jax.ShapeDtypeStruct((B, 128, N), jnp.float32),
        scratch_shapes=[
            pltpu.VMEM((64, 1), jnp.float32),
            pltpu.VMEM((64, 1), jnp.float32),
            pltpu.VMEM((128, 1), jnp.float32),
            pltpu.VMEM((128, 1), jnp.float32),
        ],
    )(xp, w1p, b1, g1, be1, w2, b2, g2, be2)


def kernel(coord, W1, b1, gamma1, beta1, W2, b2, gamma2, beta2):
    xs = coord[:, :, 0]
    ys = coord[:, :, 1]
    zs = coord[:, :, 2]
    far0 = jax.random.randint(jax.random.key(1), (B,), 0, N, dtype=jnp.int32)

    centroids = _fps(xs, ys, zs, far0.reshape(B, 1))

    d = _dist(xs, ys, zs, centroids.T)
    gi = _ball(d.reshape(B * S, N))
    group_idx = gi.reshape(B, S, K)

    xcm = jnp.transpose(coord, (2, 0, 1)).reshape(3, B * N)
    xp = jnp.concatenate([xcm, jnp.zeros((5, B * N), jnp.float32)], axis=0)
    w1p = jnp.concatenate([W1, jnp.zeros((64, 5), jnp.float32)], axis=1)
    H = _mlp(xp, w1p, b1.reshape(64, 1), gamma1.reshape(64, 1),
             beta1.reshape(64, 1), W2, b2.reshape(128, 1),
             gamma2.reshape(128, 1), beta2.reshape(128, 1))
    h = H.reshape(B * N, 128)
    return (centroids, group_idx, h)


# speedup vs baseline: 1.3773x; 1.0956x over previous
import functools

import jax
import jax.numpy as jnp
from jax import lax
from jax.experimental import pallas as pl
from jax.experimental.pallas import tpu as pltpu
from jax.experimental.pallas import tpu_sc as plsc

B, N, S, K = 8, 4096, 512, 64
R2 = 0.2 * 0.2
NPH = 16
BLK = (B * N) // NPH


def _fps_body(xs_ref, ys_ref, zs_ref, far0_ref, out_ref, dist_ref, cent_ref):
    x = xs_ref[...]
    y = ys_ref[...]
    z = zs_ref[...]
    iota = lax.broadcasted_iota(jnp.int32, (B, N), 1)
    iota_s = lax.broadcasted_iota(jnp.int32, (B, S), 1)
    iota_l = lax.broadcasted_iota(jnp.int32, (B, 128), 1)
    dist_ref[...] = jnp.full((B, N), 1e10, jnp.float32)

    m0 = iota == far0_ref[...]
    cx0 = jnp.sum(jnp.where(m0, x, 0.0), axis=1, keepdims=True)
    cy0 = jnp.sum(jnp.where(m0, y, 0.0), axis=1, keepdims=True)
    cz0 = jnp.sum(jnp.where(m0, z, 0.0), axis=1, keepdims=True)

    def comb(a, b):
        keep = a[0] >= b[0]
        return tuple(jnp.where(keep, ai, bi) for ai, bi in zip(a, b))

    def comb_ix(a, b):
        keep = (a[0] > b[0]) | ((a[0] == b[0]) & (a[1] < b[1]))
        return tuple(jnp.where(keep, ai, bi) for ai, bi in zip(a, b))

    def step(i, carry):
        far, cx, cy, cz = carry
        cent_ref[...] = jnp.where(iota_s == i, far, cent_ref[...])
        best = None
        for j in range(N // 128):
            sl = slice(j * 128, (j + 1) * 128)
            xj = x[:, sl]
            yj = y[:, sl]
            zj = z[:, sl]
            dxj = xj - cx
            dyj = yj - cy
            dzj = zj - cz
            dj = dxj * dxj + dyj * dyj + dzj * dzj
            daj = jnp.minimum(dist_ref[:, sl], dj)
            dist_ref[:, sl] = daj
            node = (daj, iota_l + (j * 128), xj, yj, zj)
            best = node if best is None else comb(best, node)
        v, ii, xx, yy, zz = best
        mxv = jnp.max(v, axis=1, keepdims=True)
        selv = v == mxv
        nf = jnp.min(jnp.where(selv, ii, N), axis=1, keepdims=True)
        win = ii == nf
        ncx = jnp.sum(jnp.where(win, xx, 0.0), axis=1, keepdims=True)
        ncy = jnp.sum(jnp.where(win, yy, 0.0), axis=1, keepdims=True)
        ncz = jnp.sum(jnp.where(win, zz, 0.0), axis=1, keepdims=True)
        return nf.astype(jnp.int32), ncx, ncy, ncz

    lax.fori_loop(0, S, step, (far0_ref[...], cx0, cy0, cz0))
    out_ref[...] = cent_ref[...]


def _fps(xs, ys, zs, far0):
    return pl.pallas_call(
        _fps_body,
        out_shape=jax.ShapeDtypeStruct((B, S), jnp.int32),
        scratch_shapes=[pltpu.VMEM((B, N), jnp.float32),
                        pltpu.VMEM((B, S), jnp.int32)],
    )(xs, ys, zs, far0)


SBLK = 128


def _dist_body(xs_ref, ys_ref, zs_ref, cent_t_ref, out_ref):
    b = pl.program_id(0)
    x = xs_ref[0, :, :]
    y = ys_ref[0, :, :]
    z = zs_ref[0, :, :]
    cb = cent_t_ref[...]
    iota_b = lax.broadcasted_iota(jnp.int32, (SBLK, B), 1)
    c = jnp.sum(jnp.where(iota_b == b, cb, 0), axis=1, keepdims=True)
    iota = lax.broadcasted_iota(jnp.int32, (SBLK, N), 1)
    oh = iota == c
    sx = jnp.sum(jnp.where(oh, x, 0.0), axis=1, keepdims=True)
    sy = jnp.sum(jnp.where(oh, y, 0.0), axis=1, keepdims=True)
    sz = jnp.sum(jnp.where(oh, z, 0.0), axis=1, keepdims=True)
    samples = jnp.concatenate([sx, sy, sz], axis=1)
    coords = jnp.concatenate([x, y, z], axis=0)
    t = -2.0 * jnp.dot(samples, coords)
    ss = sx * sx + sy * sy + sz * sz
    pp = x * x + y * y + z * z
    out_ref[0, :, :] = (t + ss) + pp


def _dist(xs, ys, zs, cent_t):
    return pl.pallas_call(
        _dist_body,
        grid=(B, S // SBLK),
        in_specs=[
            pl.BlockSpec((1, 1, N), lambda b, s: (b, 0, 0)),
            pl.BlockSpec((1, 1, N), lambda b, s: (b, 0, 0)),
            pl.BlockSpec((1, 1, N), lambda b, s: (b, 0, 0)),
            pl.BlockSpec((SBLK, B), lambda b, s: (s, 0)),
        ],
        out_specs=pl.BlockSpec((1, SBLK, N), lambda b, s: (b, s, 0)),
        out_shape=jax.ShapeDtypeStruct((B, S, N), jnp.float32),
    )(xs.reshape(B, 1, N), ys.reshape(B, 1, N), zs.reshape(B, 1, N), cent_t)


NW = 32
ROWS_PER_W = (B * S) // NW
NCHUNK = N // 16


GR = 8
NGRP = ROWS_PER_W // GR


def _ball_body(d_hbm, gi_hbm, buf_a, buf_b, outbuf, sem_a, sem_b):
    wid = lax.axis_index("s") * 2 + lax.axis_index("c")
    base = wid * ROWS_PER_W
    iota16 = lax.iota(jnp.int32, 16)

    def issue(g, buf, sem):
        pltpu.make_async_copy(d_hbm.at[pl.ds(base + g * GR, GR)], buf, sem).start()

    def wait(buf, sem):
        pltpu.make_async_copy(d_hbm.at[pl.ds(base, GR)], buf, sem).wait()

    SEG = 64
    NSEG = NCHUNK // SEG

    def process(buf, g):
        def row_body(ro, carry):
            ro_s = jnp.full((16,), ro, jnp.int32)

            def seg_step(c):
                seg, cur0 = c
                sbase = seg * (SEG * 16)

                @plsc.parallel_loop(0, SEG, unroll=8, carry=cur0)
                def scan(k, cur):
                    v = buf[ro, pl.ds(sbase + k * 16, 16)]
                    mask = v <= R2
                    mi = mask.astype(jnp.int32)
                    incl = plsc.cumsum(mi)
                    p = cur + incl - 1
                    wm = mask & (p < K)
                    idx = sbase + k * 16 + iota16
                    plsc.store_scatter(outbuf, [ro_s, p], idx, mask=wm)
                    return cur + plsc.all_reduce_population_count(mask)

                return seg + 1, scan

            def seg_cond(c):
                seg, cur = c
                return (seg < NSEG) & (jnp.max(cur) < K)

            _, cur = lax.while_loop(seg_cond, seg_step,
                                    (0, jnp.zeros((16,), jnp.int32)))
            cur_s = jnp.max(cur)
            f0 = outbuf[ro, pl.ds(0, 16)]
            first_s = jnp.min(jnp.where(iota16 == 0, f0, N))
            first_s = jnp.where(cur_s > 0, first_s, N)
            for j in range(K // 16):
                sl = j * 16 + iota16
                vals = outbuf[ro, pl.ds(j * 16, 16)]
                outbuf[ro, pl.ds(j * 16, 16)] = jnp.where(sl < cur_s, vals, first_s)
            return carry

        lax.fori_loop(0, GR, row_body, 0)
        pltpu.sync_copy(outbuf, gi_hbm.at[pl.ds(base + g * GR, GR)])

    issue(0, buf_a, sem_a)

    def pair(gg, carry):
        ga = 2 * gg
        issue(ga + 1, buf_b, sem_b)
        wait(buf_a, sem_a)
        process(buf_a, ga)

        @pl.when(gg < NGRP // 2 - 1)
        def _():
            issue(ga + 2, buf_a, sem_a)

        wait(buf_b, sem_b)
        process(buf_b, ga + 1)
        return carry

    lax.fori_loop(0, NGRP // 2, pair, 0)


def _ball(d2):
    mesh = plsc.VectorSubcoreMesh(core_axis_name="c", subcore_axis_name="s")
    f = pl.kernel(
        _ball_body,
        out_type=jax.ShapeDtypeStruct((B * S, K), jnp.int32),
        mesh=mesh,
        compiler_params=pltpu.CompilerParams(needs_layout_passes=False),
        scratch_types=[
            pltpu.VMEM((GR, N), jnp.float32),
            pltpu.VMEM((GR, N), jnp.float32),
            pltpu.VMEM((GR, K), jnp.int32),
            pltpu.SemaphoreType.DMA,
            pltpu.SemaphoreType.DMA,
        ],
    )
    return f(d2)


def _mlp_body(xp_ref, w1_ref, b1_ref, g1_ref, be1_ref, w2_ref, b2_ref,
              g2_ref, be2_ref, out_ref, s1, s2, t1, t2):
    ph = pl.program_id(0)
    i = pl.program_id(1)
    eps = jnp.float32(1e-5)
    inv = jnp.float32(1.0 / (B * N))
    h1 = jnp.dot(w1_ref[...], xp_ref[...]) + b1_ref[...]

    @pl.when((ph == 0) & (i == 0))
    def _():
        s1[...] = jnp.zeros_like(s1)
        s2[...] = jnp.zeros_like(s2)
        t1[...] = jnp.zeros_like(t1)
        t2[...] = jnp.zeros_like(t2)

    @pl.when(ph == 0)
    def _():
        s1[...] += jnp.sum(h1, axis=1, keepdims=True)
        s2[...] += jnp.sum(h1 * h1, axis=1, keepdims=True)

    @pl.when(ph >= 1)
    def _():
        m1 = s1[...] * inv
        v1 = s2[...] * inv - m1 * m1
        yy = g1_ref[...] * (h1 - m1) / jnp.sqrt(v1 + eps) + be1_ref[...]
        y = jnp.maximum(yy, 0.0)
        h2 = jnp.dot(w2_ref[...], y) + b2_ref[...]

        @pl.when(ph == 1)
        def _():
            t1[...] += jnp.sum(h2, axis=1, keepdims=True)
            t2[...] += jnp.sum(h2 * h2, axis=1, keepdims=True)

        @pl.when(ph == 2)
        def _():
            m2 = t1[...] * inv
            v2 = t2[...] * inv - m2 * m2
            z = g2_ref[...] * (h2 - m2) / jnp.sqrt(v2 + eps) + be2_ref[...]
            out_ref[0, :, :] = jnp.maximum(z, 0.0)


def _mlp(xp, w1p, b1, g1, be1, w2, b2, g2, be2):
    full = lambda shape: pl.BlockSpec(shape, lambda ph, i: (0, 0))
    return pl.pallas_call(
        _mlp_body,
        grid=(3, NPH),
        in_specs=[
            pl.BlockSpec((8, BLK), lambda ph, i: (0, i)),
            full((64, 8)), full((64, 1)), full((64, 1)), full((64, 1)),
            full((128, 64)), full((128, 1)), full((128, 1)), full((128, 1)),
        ],
        out_specs=pl.BlockSpec((1, 128, BLK),
                               lambda ph, i: (i // (N // BLK), 0, i % (N // BLK))),
        out_shape=jax.ShapeDtypeStruct((B, 128, N), jnp.float32),
        scratch_shapes=[
            pltpu.VMEM((64, 1), jnp.float32),
            pltpu.VMEM((64, 1), jnp.float32),
            pltpu.VMEM((128, 1), jnp.float32),
            pltpu.VMEM((128, 1), jnp.float32),
        ],
    )(xp, w1p, b1, g1, be1, w2, b2, g2, be2)


def kernel(coord, W1, b1, gamma1, beta1, W2, b2, gamma2, beta2):
    xs = coord[:, :, 0]
    ys = coord[:, :, 1]
    zs = coord[:, :, 2]
    far0 = jax.random.randint(jax.random.key(1), (B,), 0, N, dtype=jnp.int32)

    centroids = _fps(xs, ys, zs, far0.reshape(B, 1))

    d = _dist(xs, ys, zs, centroids.T)
    gi = _ball(d.reshape(B * S, N))
    group_idx = gi.reshape(B, S, K)

    xcm = jnp.transpose(coord, (2, 0, 1)).reshape(3, B * N)
    xp = jnp.concatenate([xcm, jnp.zeros((5, B * N), jnp.float32)], axis=0)
    w1p = jnp.concatenate([W1, jnp.zeros((64, 5), jnp.float32)], axis=1)
    H = _mlp(xp, w1p, b1.reshape(64, 1), gamma1.reshape(64, 1),
             beta1.reshape(64, 1), W2, b2.reshape(128, 1),
             gamma2.reshape(128, 1), beta2.reshape(128, 1))
    h = H.reshape(B * N, 128)
    return (centroids, group_idx, h)


# speedup vs baseline: 1.3801x; 1.0020x over previous
import jax
import jax.numpy as jnp
from jax import lax
from jax.experimental import pallas as pl
from jax.experimental.pallas import tpu as pltpu
from jax.experimental.pallas import tpu_sc as plsc

B, N, S, K = 8, 4096, 512, 64
R2 = 0.2 * 0.2
NPH = 16
BLK = (B * N) // NPH


def _fps_body(xs_ref, ys_ref, zs_ref, far0_ref, out_ref, dist_ref, cent_ref):
    x = xs_ref[...]
    y = ys_ref[...]
    z = zs_ref[...]
    iota = lax.broadcasted_iota(jnp.int32, (B, N), 1)
    iota_s = lax.broadcasted_iota(jnp.int32, (B, S), 1)
    iota_l = lax.broadcasted_iota(jnp.int32, (B, 128), 1)
    dist_ref[...] = jnp.full((B, N), 1e10, jnp.float32)

    m0 = iota == far0_ref[...]
    cx0 = jnp.sum(jnp.where(m0, x, 0.0), axis=1, keepdims=True)
    cy0 = jnp.sum(jnp.where(m0, y, 0.0), axis=1, keepdims=True)
    cz0 = jnp.sum(jnp.where(m0, z, 0.0), axis=1, keepdims=True)

    def comb(a, b):
        keep = a[0] >= b[0]
        return tuple(jnp.where(keep, ai, bi) for ai, bi in zip(a, b))

    def step(i, carry):
        far, cx, cy, cz = carry
        cent_ref[...] = jnp.where(iota_s == i, far, cent_ref[...])
        best = None
        for j in range(N // 128):
            sl = slice(j * 128, (j + 1) * 128)
            xj = x[:, sl]
            yj = y[:, sl]
            zj = z[:, sl]
            dxj = xj - cx
            dyj = yj - cy
            dzj = zj - cz
            dj = dxj * dxj + dyj * dyj + dzj * dzj
            daj = jnp.minimum(dist_ref[:, sl], dj)
            dist_ref[:, sl] = daj
            node = (daj, iota_l + (j * 128), xj, yj, zj)
            best = node if best is None else comb(best, node)
        v, ii, xx, yy, zz = best
        mxv = jnp.max(v, axis=1, keepdims=True)
        selv = v == mxv
        nf = jnp.min(jnp.where(selv, ii, N), axis=1, keepdims=True)
        win = ii == nf
        ncx = jnp.sum(jnp.where(win, xx, 0.0), axis=1, keepdims=True)
        ncy = jnp.sum(jnp.where(win, yy, 0.0), axis=1, keepdims=True)
        ncz = jnp.sum(jnp.where(win, zz, 0.0), axis=1, keepdims=True)
        return nf.astype(jnp.int32), ncx, ncy, ncz

    lax.fori_loop(0, S, step, (far0_ref[...], cx0, cy0, cz0))
    out_ref[...] = cent_ref[...]


def _fps(xs, ys, zs, far0):
    return pl.pallas_call(
        _fps_body,
        out_shape=jax.ShapeDtypeStruct((B, S), jnp.int32),
        scratch_shapes=[pltpu.VMEM((B, N), jnp.float32),
                        pltpu.VMEM((B, S), jnp.int32)],
    )(xs, ys, zs, far0)


SBLK = 128


def _dist_body(xs_ref, ys_ref, zs_ref, cent_t_ref, out_ref):
    b = pl.program_id(0)
    x = xs_ref[0, :, :]
    y = ys_ref[0, :, :]
    z = zs_ref[0, :, :]
    cb = cent_t_ref[...]
    iota_b = lax.broadcasted_iota(jnp.int32, (SBLK, B), 1)
    c = jnp.sum(jnp.where(iota_b == b, cb, 0), axis=1, keepdims=True)
    iota = lax.broadcasted_iota(jnp.int32, (SBLK, N), 1)
    oh = iota == c
    sx = jnp.sum(jnp.where(oh, x, 0.0), axis=1, keepdims=True)
    sy = jnp.sum(jnp.where(oh, y, 0.0), axis=1, keepdims=True)
    sz = jnp.sum(jnp.where(oh, z, 0.0), axis=1, keepdims=True)
    samples = jnp.concatenate([sx, sy, sz], axis=1)
    coords = jnp.concatenate([x, y, z], axis=0)
    t = -2.0 * jnp.dot(samples, coords)
    ss = sx * sx + sy * sy + sz * sz
    pp = x * x + y * y + z * z
    out_ref[0, :, :] = (t + ss) + pp


def _dist(xs, ys, zs, cent_t):
    return pl.pallas_call(
        _dist_body,
        grid=(B, S // SBLK),
        in_specs=[
            pl.BlockSpec((1, 1, N), lambda b, s: (b, 0, 0)),
            pl.BlockSpec((1, 1, N), lambda b, s: (b, 0, 0)),
            pl.BlockSpec((1, 1, N), lambda b, s: (b, 0, 0)),
            pl.BlockSpec((SBLK, B), lambda b, s: (s, 0)),
        ],
        out_specs=pl.BlockSpec((1, SBLK, N), lambda b, s: (b, s, 0)),
        out_shape=jax.ShapeDtypeStruct((B, S, N), jnp.float32),
    )(xs.reshape(B, 1, N), ys.reshape(B, 1, N), zs.reshape(B, 1, N), cent_t)


NW = 32
ROWS_PER_W = (B * S) // NW
NCHUNK = N // 16


GR = 8
NGRP = ROWS_PER_W // GR


def _ball_body(d_hbm, gi_hbm, buf_a, buf_b, outbuf, sem_a, sem_b):
    wid = lax.axis_index("s") * 2 + lax.axis_index("c")
    base = wid * ROWS_PER_W
    iota16 = lax.iota(jnp.int32, 16)

    def issue(g, buf, sem):
        pltpu.make_async_copy(d_hbm.at[pl.ds(base + g * GR, GR)], buf, sem).start()

    def wait(buf, sem):
        pltpu.make_async_copy(d_hbm.at[pl.ds(base, GR)], buf, sem).wait()

    SEG = 64
    NSEG = NCHUNK // SEG

    def process(buf, g):
        def row_body(ro, carry):
            ro_s = jnp.full((16,), ro, jnp.int32)

            def seg_step(c):
                seg, cur0 = c
                sbase = seg * (SEG * 16)

                @plsc.parallel_loop(0, SEG, unroll=8, carry=cur0)
                def scan(k, cur):
                    v = buf[ro, pl.ds(sbase + k * 16, 16)]
                    mask = v <= R2
                    mi = mask.astype(jnp.int32)
                    incl = plsc.cumsum(mi)
                    p = cur + incl - 1
                    wm = mask & (p < K)
                    idx = sbase + k * 16 + iota16
                    plsc.store_scatter(outbuf, [ro_s, p], idx, mask=wm)
                    return cur + plsc.all_reduce_population_count(mask)

                return seg + 1, scan

            def seg_cond(c):
                seg, cur = c
                return (seg < NSEG) & (jnp.max(cur) < K)

            _, cur = lax.while_loop(seg_cond, seg_step,
                                    (0, jnp.zeros((16,), jnp.int32)))
            cur_s = jnp.max(cur)
            f0 = outbuf[ro, pl.ds(0, 16)]
            first_s = jnp.min(jnp.where(iota16 == 0, f0, N))
            first_s = jnp.where(cur_s > 0, first_s, N)
            for j in range(K // 16):
                sl = j * 16 + iota16
                vals = outbuf[ro, pl.ds(j * 16, 16)]
                outbuf[ro, pl.ds(j * 16, 16)] = jnp.where(sl < cur_s, vals, first_s)
            return carry

        lax.fori_loop(0, GR, row_body, 0)
        pltpu.sync_copy(outbuf, gi_hbm.at[pl.ds(base + g * GR, GR)])

    issue(0, buf_a, sem_a)

    def pair(gg, carry):
        ga = 2 * gg
        issue(ga + 1, buf_b, sem_b)
        wait(buf_a, sem_a)
        process(buf_a, ga)

        @pl.when(gg < NGRP // 2 - 1)
        def _():
            issue(ga + 2, buf_a, sem_a)

        wait(buf_b, sem_b)
        process(buf_b, ga + 1)
        return carry

    lax.fori_loop(0, NGRP // 2, pair, 0)


def _ball(d2):
    mesh = plsc.VectorSubcoreMesh(core_axis_name="c", subcore_axis_name="s")
    f = pl.kernel(
        _ball_body,
        out_type=jax.ShapeDtypeStruct((B * S, K), jnp.int32),
        mesh=mesh,
        compiler_params=pltpu.CompilerParams(needs_layout_passes=False),
        scratch_types=[
            pltpu.VMEM((GR, N), jnp.float32),
            pltpu.VMEM((GR, N), jnp.float32),
            pltpu.VMEM((GR, K), jnp.int32),
            pltpu.SemaphoreType.DMA,
            pltpu.SemaphoreType.DMA,
        ],
    )
    return f(d2)


def _mlp_body(xp_ref, w1_ref, b1_ref, g1_ref, be1_ref, w2_ref, b2_ref,
              g2_ref, be2_ref, out_ref, s1, s2, t1, t2):
    ph = pl.program_id(0)
    i = pl.program_id(1)
    eps = jnp.float32(1e-5)
    inv = jnp.float32(1.0 / (B * N))
    h1 = jnp.dot(w1_ref[...], xp_ref[...]) + b1_ref[...]

    @pl.when((ph == 0) & (i == 0))
    def _():
        s1[...] = jnp.zeros_like(s1)
        s2[...] = jnp.zeros_like(s2)
        t1[...] = jnp.zeros_like(t1)
        t2[...] = jnp.zeros_like(t2)

    @pl.when(ph == 0)
    def _():
        s1[...] += jnp.sum(h1, axis=1, keepdims=True)
        s2[...] += jnp.sum(h1 * h1, axis=1, keepdims=True)

    @pl.when(ph >= 1)
    def _():
        m1 = s1[...] * inv
        v1 = s2[...] * inv - m1 * m1
        yy = g1_ref[...] * (h1 - m1) / jnp.sqrt(v1 + eps) + be1_ref[...]
        y = jnp.maximum(yy, 0.0)
        h2 = jnp.dot(w2_ref[...], y) + b2_ref[...]

        @pl.when(ph == 1)
        def _():
            t1[...] += jnp.sum(h2, axis=1, keepdims=True)
            t2[...] += jnp.sum(h2 * h2, axis=1, keepdims=True)

        @pl.when(ph == 2)
        def _():
            m2 = t1[...] * inv
            v2 = t2[...] * inv - m2 * m2
            z = g2_ref[...] * (h2 - m2) / jnp.sqrt(v2 + eps) + be2_ref[...]
            out_ref[0, :, :] = jnp.maximum(z, 0.0)


def _mlp(xp, w1p, b1, g1, be1, w2, b2, g2, be2):
    full = lambda shape: pl.BlockSpec(shape, lambda ph, i: (0, 0))
    return pl.pallas_call(
        _mlp_body,
        grid=(3, NPH),
        in_specs=[
            pl.BlockSpec((8, BLK), lambda ph, i: (0, i)),
            full((64, 8)), full((64, 1)), full((64, 1)), full((64, 1)),
            full((128, 64)), full((128, 1)), full((128, 1)), full((128, 1)),
        ],
        out_specs=pl.BlockSpec((1, 128, BLK),
                               lambda ph, i: (i // (N // BLK), 0, i % (N // BLK))),
        out_shape=jax.ShapeDtypeStruct((B, 128, N), jnp.float32),
        scratch_shapes=[
            pltpu.VMEM((64, 1), jnp.float32),
            pltpu.VMEM((64, 1), jnp.float32),
            pltpu.VMEM((128, 1), jnp.float32),
            pltpu.VMEM((128, 1), jnp.float32),
        ],
    )(xp, w1p, b1, g1, be1, w2, b2, g2, be2)


def kernel(coord, W1, b1, gamma1, beta1, W2, b2, gamma2, beta2):
    xs = coord[:, :, 0]
    ys = coord[:, :, 1]
    zs = coord[:, :, 2]
    far0 = jax.random.randint(jax.random.key(1), (B,), 0, N, dtype=jnp.int32)

    centroids = _fps(xs, ys, zs, far0.reshape(B, 1))

    d = _dist(xs, ys, zs, centroids.T)
    gi = _ball(d.reshape(B * S, N))
    group_idx = gi.reshape(B, S, K)

    xcm = jnp.transpose(coord, (2, 0, 1)).reshape(3, B * N)
    xp = jnp.concatenate([xcm, jnp.zeros((5, B * N), jnp.float32)], axis=0)
    w1p = jnp.concatenate([W1, jnp.zeros((64, 5), jnp.float32)], axis=1)
    H = _mlp(xp, w1p, b1.reshape(64, 1), gamma1.reshape(64, 1),
             beta1.reshape(64, 1), W2, b2.reshape(128, 1),
             gamma2.reshape(128, 1), beta2.reshape(128, 1))
    h = H.reshape(B * N, 128)
    return (centroids, group_idx, h)
